# baseline (speedup 1.0000x reference)
import functools

import jax
import jax.numpy as jnp
from jax import lax
from jax.experimental import pallas as pl
from jax.experimental.pallas import tpu as pltpu
from jax.experimental.pallas import tpu_sc as plsc

N = 100000
CIN = 32
COUT = 32
KV = 27
BASE = 130

NW = 32
CHUNK = 128
NQ = KV * N
CPW = (NQ + NW * CHUNK - 1) // (NW * CHUNK)
NQ_PAD = NW * CHUNK * CPW


def _deltas():
    ds = []
    for z in range(3):
        for y in range(3):
            for x in range(3):
                ds.append((x - 1) * BASE * BASE + (y - 1) * BASE + (z - 1))
    return jnp.asarray(ds, dtype=jnp.int32)


def _gather_body(feats_hbm, idx_hbm, out_hbm, idx_v, rows_v, sem):
    wid = lax.axis_index("s") * 2 + lax.axis_index("c")

    def step(j, _):
        pltpu.sync_copy(idx_hbm.at[wid, j], idx_v)
        pltpu.async_copy(feats_hbm.at[idx_v], rows_v, sem).wait()
        pltpu.sync_copy(rows_v, out_hbm.at[pl.ds((wid * CPW + j) * CHUNK, CHUNK)])
        return _

    lax.fori_loop(0, CPW, step, 0)


@jax.jit
def _sc_gather(feats_pad, idx3):
    mesh = plsc.VectorSubcoreMesh(core_axis_name="c", subcore_axis_name="s")
    return pl.kernel(
        _gather_body,
        mesh=mesh,
        compiler_params=pltpu.CompilerParams(use_tc_tiling_on_sc=False),
        out_type=jax.ShapeDtypeStruct((NQ_PAD, CIN), jnp.float32),
        scratch_types=[
            pltpu.VMEM((CHUNK,), jnp.int32),
            pltpu.VMEM((CHUNK, CIN), jnp.float32),
            pltpu.SemaphoreType.DMA,
        ],
    )(feats_pad, idx3)


BR = 1000


def _mm_body(g_ref, w_ref, b_ref, o_ref):
    o_ref[...] = (
        jnp.dot(g_ref[...], w_ref[...], preferred_element_type=jnp.float32)
        + b_ref[...]
    )


@jax.jit
def _tc_matmul(gathered, w2, bias2):
    return pl.pallas_call(
        _mm_body,
        grid=(N // BR,),
        in_specs=[
            pl.BlockSpec((BR, KV * CIN), lambda i: (i, 0)),
            pl.BlockSpec((KV * CIN, COUT), lambda i: (0, 0)),
            pl.BlockSpec((1, COUT), lambda i: (0, 0)),
        ],
        out_specs=pl.BlockSpec((BR, COUT), lambda i: (i, 0)),
        out_shape=jax.ShapeDtypeStruct((N, COUT), jnp.float32),
    )(gathered, w2, bias2)


def kernel(feats, coords, weight, bias):
    c = coords + 1
    keys = ((c[:, 0] * BASE + c[:, 1]) * BASE + c[:, 2]) * BASE + c[:, 3]

    order = jnp.argsort(keys)
    skeys = keys[order]
    tkeys = keys[None, :] + _deltas()[:, None]
    pos = jnp.searchsorted(skeys, tkeys.ravel()).reshape(KV, N)
    pos_c = jnp.clip(pos, 0, N - 1)
    match = skeys[pos_c] == tkeys
    nbr = jnp.where(match, order[pos_c], N)

    idx_flat = nbr.T.reshape(-1)
    idx_flat = jnp.concatenate(
        [idx_flat, jnp.full((NQ_PAD - NQ,), N, dtype=jnp.int32)])
    idx3 = idx_flat.reshape(NW, CPW, CHUNK)

    feats_pad = jnp.concatenate(
        [feats, jnp.zeros((1, CIN), dtype=feats.dtype)])

    gathered = _sc_gather(feats_pad, idx3)[:NQ].reshape(N, KV * CIN)
    w2 = weight.reshape(KV * CIN, COUT)
    return _tc_matmul(gathered, w2, bias.reshape(1, COUT))
